# baseline (speedup 1.0000x reference)
import jax
import jax.numpy as jnp
from jax import lax
from jax.experimental import pallas as pl
from jax.experimental.pallas import tpu as pltpu
from jax.experimental.pallas import tpu_sc as plsc

NC, NS, L = 2, 16, 16
NW = NC * NS
B = 16384
RPW = B // NW
CHUNKS = RPW // L

_SQRT_MAGIC = 0x1FBD1DF5


def _sqrt16(q):
    qi = lax.bitcast_convert_type(q, jnp.int32)
    y = lax.bitcast_convert_type(
        _SQRT_MAGIC + lax.shift_right_arithmetic(qi, 1), jnp.float32)
    y = 0.5 * (y + q / y)
    y = 0.5 * (y + q / y)
    return y


def _sc_body(idx_hbm, g1_hbm, g0_hbm, et_hbm, wm_hbm, out_hbm,
             idx_v, g1_v, g0_v, et_v, wm_v, out_v):
    wid = lax.axis_index("s") * NC + lax.axis_index("c")
    base = wid * RPW
    pltpu.sync_copy(idx_hbm.at[pl.ds(base, RPW), :], idx_v)
    pltpu.sync_copy(g1_hbm.at[pl.ds(base, RPW), :], g1_v)
    pltpu.sync_copy(g0_hbm.at[pl.ds(base, RPW), :], g0_v)
    pltpu.sync_copy(et_hbm, et_v)
    pltpu.sync_copy(wm_hbm, wm_v)

    iota = lax.iota(jnp.int32, L)
    col = [jnp.full((L,), j, jnp.int32) for j in range(5)]
    zeros, ones = col[0], col[1]
    wm0 = wm_v[0, :]
    wm1 = wm_v[1, :]

    def chunk(i, carry):
        row = i * L + iota
        w1a = plsc.load_gather(g1_v, [row, zeros])
        w1b = plsc.load_gather(g1_v, [row, ones])
        w0a = plsc.load_gather(g0_v, [row, zeros])
        w0b = plsc.load_gather(g0_v, [row, ones])
        c0 = w0a * w1a
        c1 = w0a * w1b
        c2 = w0b * w1a
        c3 = w0b * w1b
        z = []
        for j in range(5):
            o = plsc.load_gather(idx_v, [row, col[j]]) * 2
            e0a = plsc.load_gather(et_v, [o])
            e0b = plsc.load_gather(et_v, [o + 1])
            e1a = plsc.load_gather(et_v, [o + 62])
            e1b = plsc.load_gather(et_v, [o + 63])
            e2a = plsc.load_gather(et_v, [o + 124])
            e2b = plsc.load_gather(et_v, [o + 125])
            e3a = plsc.load_gather(et_v, [o + 186])
            e3b = plsc.load_gather(et_v, [o + 187])
            z.append((c0 * e0a + c1 * e1a + c2 * e2a + c3 * e3a,
                      c0 * e0b + c1 * e1b + c2 * e2b + c3 * e3b))
        qa, qb = z[0]
        s = []
        for r in range(1, 5):
            dx = qa - z[r][0]
            dy = qb - z[r][1]
            s.append(jnp.exp(-10.0 * _sqrt16(wm0 * dx * dx + wm1 * dy * dy)))
        inv = 1.0 / (s[0] + s[1] + s[2] + s[3])
        for r in range(4):
            plsc.store_scatter(out_v, [row, col[r]], s[r] * inv)
        return carry

    lax.fori_loop(0, CHUNKS, chunk, 0)
    pltpu.sync_copy(out_v, out_hbm.at[pl.ds(base, RPW), :])


_rank_sc = pl.kernel(
    _sc_body,
    out_type=jax.ShapeDtypeStruct((B, 4), jnp.float32),
    mesh=plsc.VectorSubcoreMesh(core_axis_name="c", subcore_axis_name="s"),
    compiler_params=pltpu.CompilerParams(
        needs_layout_passes=False, use_tc_tiling_on_sc=False,
        skip_device_barrier=True),
    scratch_types=[
        pltpu.VMEM((RPW, 5), jnp.int32),
        pltpu.VMEM((RPW, 2), jnp.float32),
        pltpu.VMEM((RPW, 2), jnp.float32),
        pltpu.VMEM((256,), jnp.float32),
        pltpu.VMEM((2, 16), jnp.float32),
        pltpu.VMEM((RPW, 4), jnp.float32),
    ],
)


def kernel(given4rank1_stimulus_set, percept_gate_weights_1,
           percept_gate_weights_0, E0, E1, E2, E3, w_mink):
    idx = given4rank1_stimulus_set.astype(jnp.int32)
    etab = jnp.pad(
        jnp.concatenate([E0.reshape(-1), E1.reshape(-1),
                         E2.reshape(-1), E3.reshape(-1)]), (0, 8))
    wm = jnp.broadcast_to(w_mink[:, None], (2, 16))
    return _rank_sc(idx, percept_gate_weights_1, percept_gate_weights_0,
                    etab, wm)


# speedup vs baseline: 1.1644x; 1.1644x over previous
import jax
import jax.numpy as jnp
from jax import lax
from jax.experimental import pallas as pl
from jax.experimental.pallas import tpu as pltpu
from jax.experimental.pallas import tpu_sc as plsc

NC, NS, L = 2, 16, 16
NW = NC * NS
B = 16384
RPW = B // NW
CHUNKS = RPW // L

_SQRT_MAGIC = 0x1FBD1DF5


def _sqrt16(q):
    qi = lax.bitcast_convert_type(q, jnp.int32)
    y = lax.bitcast_convert_type(
        _SQRT_MAGIC + lax.shift_right_arithmetic(qi, 1), jnp.float32)
    y = 0.5 * (y + q / y)
    y = 0.5 * (y + q / y)
    return y


def _sc_body(idx_hbm, g1_hbm, g0_hbm, et_hbm, wm_hbm, out_hbm,
             idx_v, g1_v, g0_v, et_v, wm_v, out_v):
    wid = lax.axis_index("s") * NC + lax.axis_index("c")
    base = wid * RPW
    pltpu.sync_copy(idx_hbm.at[pl.ds(base * 5, RPW * 5)], idx_v)
    pltpu.sync_copy(g1_hbm.at[pl.ds(base * 2, RPW * 2)], g1_v)
    pltpu.sync_copy(g0_hbm.at[pl.ds(base * 2, RPW * 2)], g0_v)
    pltpu.sync_copy(et_hbm, et_v)
    pltpu.sync_copy(wm_hbm, wm_v)

    iota = lax.iota(jnp.int32, L)
    wm0 = wm_v[pl.ds(0, L)]
    wm1 = wm_v[pl.ds(L, L)]

    def chunk(i, carry):
        row = i * L + iota
        row2 = row * 2
        w1a = plsc.load_gather(g1_v, [row2])
        w1b = plsc.load_gather(g1_v, [row2 + 1])
        w0a = plsc.load_gather(g0_v, [row2])
        w0b = plsc.load_gather(g0_v, [row2 + 1])
        c0 = w0a * w1a
        c1 = w0a * w1b
        c2 = w0b * w1a
        c3 = w0b * w1b
        row5 = row * 5
        z = []
        for j in range(5):
            o = plsc.load_gather(idx_v, [row5 + j]) * 2
            e0a = plsc.load_gather(et_v, [o])
            e0b = plsc.load_gather(et_v, [o + 1])
            e1a = plsc.load_gather(et_v, [o + 62])
            e1b = plsc.load_gather(et_v, [o + 63])
            e2a = plsc.load_gather(et_v, [o + 124])
            e2b = plsc.load_gather(et_v, [o + 125])
            e3a = plsc.load_gather(et_v, [o + 186])
            e3b = plsc.load_gather(et_v, [o + 187])
            z.append((c0 * e0a + c1 * e1a + c2 * e2a + c3 * e3a,
                      c0 * e0b + c1 * e1b + c2 * e2b + c3 * e3b))
        qa, qb = z[0]
        s = []
        for r in range(1, 5):
            dx = qa - z[r][0]
            dy = qb - z[r][1]
            s.append(jnp.exp(-10.0 * _sqrt16(wm0 * dx * dx + wm1 * dy * dy)))
        inv = 1.0 / (s[0] + s[1] + s[2] + s[3])
        row4 = row * 4
        for r in range(4):
            plsc.store_scatter(out_v, [row4 + r], s[r] * inv)
        return carry

    lax.fori_loop(0, CHUNKS, chunk, 0)
    pltpu.sync_copy(out_v, out_hbm.at[pl.ds(base * 4, RPW * 4)])


_rank_sc = pl.kernel(
    _sc_body,
    out_type=jax.ShapeDtypeStruct((B * 4,), jnp.float32),
    mesh=plsc.VectorSubcoreMesh(core_axis_name="c", subcore_axis_name="s"),
    compiler_params=pltpu.CompilerParams(
        needs_layout_passes=False, use_tc_tiling_on_sc=False),
    scratch_types=[
        pltpu.VMEM((RPW * 5,), jnp.int32),
        pltpu.VMEM((RPW * 2,), jnp.float32),
        pltpu.VMEM((RPW * 2,), jnp.float32),
        pltpu.VMEM((256,), jnp.float32),
        pltpu.VMEM((32,), jnp.float32),
        pltpu.VMEM((RPW * 4,), jnp.float32),
    ],
)


def kernel(given4rank1_stimulus_set, percept_gate_weights_1,
           percept_gate_weights_0, E0, E1, E2, E3, w_mink):
    idx = given4rank1_stimulus_set.astype(jnp.int32).reshape(-1)
    g1 = percept_gate_weights_1.reshape(-1)
    g0 = percept_gate_weights_0.reshape(-1)
    etab = jnp.pad(
        jnp.concatenate([E0.reshape(-1), E1.reshape(-1),
                         E2.reshape(-1), E3.reshape(-1)]), (0, 8))
    wm = jnp.broadcast_to(w_mink[:, None], (2, 16)).reshape(-1)
    out = _rank_sc(idx, g1, g0, etab, wm)
    return out.reshape(B, 4)
